# baseline (speedup 1.0000x reference)
import functools

import jax
import jax.numpy as jnp
from jax import lax
from jax.experimental import pallas as pl
from jax.experimental.pallas import tpu as pltpu
from jax.experimental.pallas import tpu_sc as plsc

_NC = 2
_NS = 16
_NW = _NC * _NS

_V = 1000000
_H = 64
_LANES = 128
_TC_TOTAL = (_V + _LANES - 1) // _LANES
_TC_FULL = (_TC_TOTAL - 1) // _NW * _NW
_PAIR_ROWS = _V // 2


def _mesh():
  return plsc.VectorSubcoreMesh(core_axis_name="c", subcore_axis_name="s",
                                num_cores=_NC, num_subcores=_NS)


def _iota16():
  return lax.iota(jnp.int32, 16)


_FPITCH = 137


def _transpose_face(face_pad, q, width):
  it = _iota16()
  hrows = [it + (16 * g) for g in range(4)]

  def step(s, _):
    batch = []
    for di in range(8):
      d = s * 8 + di
      dcol = jnp.broadcast_to(d, (16,))
      for g in range(4):
        v = plsc.load_gather(face_pad, [hrows[g], dcol])
        batch.append((v, d, g))
    for v, d, g in batch:
      q[d >> 1, pl.ds(64 * (d & 1) + 16 * g, 16)] = v
    return ()

  lax.fori_loop(0, width // 8, step, ())


def _make_k1():
  per_w = _TC_FULL // _NW

  @functools.partial(
      pl.kernel,
      out_type=jax.ShapeDtypeStruct((_PAIR_ROWS, _LANES), jnp.float32),
      mesh=_mesh(),
      scratch_types=[
          pltpu.VMEM((2, _H, _FPITCH), jnp.float32),
          pltpu.VMEM((2, _H, _LANES), jnp.float32),
          [pltpu.SemaphoreType.DMA] * 2,
          [pltpu.SemaphoreType.DMA] * 2,
      ],
      compiler_params=pltpu.CompilerParams(needs_layout_passes=False, disable_bounds_checks=True),
  )
  def k1(embT, tail_pairs, out, face_v, q_v, fsems, qsems):
    wid = lax.axis_index("s") * _NC + lax.axis_index("c")

    def tc_of(k):
      return k * _NW + wid

    def face_start(k, b):
      pltpu.async_copy(embT.at[:, pl.ds(tc_of(k) * _LANES, _LANES)],
                       face_v.at[b, :, pl.ds(0, _LANES)], fsems[b])

    def face_wait(k, b):
      pltpu.make_async_copy(embT.at[:, pl.ds(tc_of(k) * _LANES, _LANES)],
                            face_v.at[b, :, pl.ds(0, _LANES)],
                            fsems[b]).wait()

    def q_start(k, b):
      off = pl.multiple_of(tc_of(k) * (_LANES // 2), 8)
      pltpu.async_copy(q_v.at[b], out.at[pl.ds(off, _H)], qsems[b])

    def q_wait(b):
      pltpu.make_async_copy(q_v.at[b], out.at[pl.ds(0, _H)], qsems[b]).wait()

    face_start(0, 0)

    def body(p, _):
      for b in range(2):
        k = p * 2 + b

        @pl.when(k + 1 < per_w)
        def _():
          face_start(k + 1, 1 - b)

        face_wait(k, b)

        @pl.when(k >= 2)
        def _():
          q_wait(b)
        _transpose_face(face_v.at[b], q_v.at[b], _LANES)
        q_start(k, b)
      return ()

    lax.fori_loop(0, per_w // 2, body, ())
    q_wait(0)
    q_wait(1)

    n_tail_full = _TC_TOTAL - 1 - _TC_FULL

    @pl.when(wid < n_tail_full)
    def _():
      tc = _TC_FULL + wid
      pltpu.sync_copy(embT.at[:, pl.ds(tc * _LANES, _LANES)],
                      face_v.at[0, :, pl.ds(0, _LANES)])
      _transpose_face(face_v.at[0], q_v.at[0], _LANES)
      pltpu.sync_copy(q_v.at[0], out.at[pl.ds(tc * (_LANES // 2), _H)])

    @pl.when(wid == n_tail_full)
    def _():
      tc = _TC_TOTAL - 1
      pltpu.sync_copy(tail_pairs, q_v.at[0, pl.ds(0, 32)])
      pltpu.sync_copy(q_v.at[0, pl.ds(0, 32)],
                      out.at[pl.ds(tc * (_LANES // 2), 32)])

  return k1


def _make_k2(total):
  per_w = total // _NW
  chunk = 64
  chunks = per_w // chunk
  nbuf = 4
  ahead = 2

  @functools.partial(
      pl.kernel,
      out_type=jax.ShapeDtypeStruct((total // 2, _LANES), jnp.float32),
      mesh=_mesh(),
      scratch_types=[
          pltpu.VMEM((per_w,), jnp.int32),
          pltpu.VMEM((per_w,), jnp.int32),
          pltpu.VMEM((per_w,), jnp.int32),
          pltpu.VMEM((nbuf, chunk, _LANES), jnp.float32),
          pltpu.VMEM((2, chunk // 2, _LANES), jnp.float32),
          [pltpu.SemaphoreType.DMA] * nbuf,
          [pltpu.SemaphoreType.DMA] * 2,
      ],
      compiler_params=pltpu.CompilerParams(needs_layout_passes=False, disable_bounds_checks=True),
  )
  def k2(table, idx_hbm, out, idx_v, pid_v, par_v, g_v, r_v, gsems, rsems):
    wid = lax.axis_index("s") * _NC + lax.axis_index("c")
    base = wid * per_w

    pltpu.sync_copy(idx_hbm.at[pl.ds(base, per_w)], idx_v)

    def prep(i, _):
      v = idx_v[pl.ds(i * 16, 16)]
      pid_v[pl.ds(i * 16, 16)] = lax.shift_right_logical(v, 1)
      par_v[pl.ds(i * 16, 16)] = lax.mul(lax.rem(v, 2), 64)
      return ()

    lax.fori_loop(0, per_w // 16, prep, ())

    def g_start(j, b):
      pltpu.async_copy(table.at[pid_v.at[pl.ds(j * chunk, chunk)]],
                       g_v.at[b], gsems[b])

    def g_wait(j, b):
      pltpu.make_async_copy(table.at[pid_v.at[pl.ds(j * chunk, chunk)]],
                            g_v.at[b], gsems[b]).wait()

    def r_start(j, b):
      off = pl.multiple_of(base // 2 + j * (chunk // 2), 8)
      pltpu.async_copy(r_v.at[b], out.at[pl.ds(off, chunk // 2)], rsems[b])

    def r_wait(b):
      pltpu.make_async_copy(r_v.at[b], out.at[pl.ds(0, chunk // 2)],
                            rsems[b]).wait()

    def select(j, gb, rb):
      for g2 in range(chunk // 16):
        vp = par_v[pl.ds(j * chunk + 16 * g2, 16)]
        for k in range(16):
          i = 16 * g2 + k
          p = vp[k]
          for g in range(4):
            v = g_v[gb, i, pl.ds(p + 16 * g, 16)]
            r_v[rb, i // 2, pl.ds(64 * (i % 2) + 16 * g, 16)] = v

    for k in range(ahead):
      g_start(k, k)

    def body(p, _):
      for b in range(nbuf):
        j = p * nbuf + b
        rb = b % 2

        @pl.when(j + ahead < chunks)
        def _():
          g_start(j + ahead, (b + ahead) % nbuf)

        g_wait(j, b)

        @pl.when(j >= 2)
        def _():
          r_wait(rb)
        select(j, b, rb)
        r_start(j, rb)
      return ()

    lax.fori_loop(0, chunks // nbuf, body, ())
    r_wait(0)
    r_wait(1)

  return k2


def kernel(inputs, embeddings):
  batch, seq = inputs.shape
  hidden = embeddings.shape[1]
  total = batch * seq
  embT = jnp.transpose(embeddings)
  idx_flat = jnp.reshape(inputs.astype(jnp.int32), (total,))
  tail_rows = (_TC_TOTAL - 1) * _LANES
  tail_pairs = jnp.reshape(embeddings[tail_rows:, :], (32, 128))
  pairs = _make_k1()(embT, tail_pairs)
  res = _make_k2(total)(pairs, idx_flat)
  return jnp.reshape(res, (batch, seq, hidden))


# speedup vs baseline: 1.6953x; 1.6953x over previous
import functools

import jax
import jax.numpy as jnp
from jax import lax
from jax.experimental import pallas as pl
from jax.experimental.pallas import tpu as pltpu
from jax.experimental.pallas import tpu_sc as plsc

_NUM_CORES = 2
_NUM_SUBCORES = 16
_NUM_WORKERS = _NUM_CORES * _NUM_SUBCORES

_NBUF = 4
_AHEAD = 3


def _make_gather(batch, seq, hidden):
  assert batch % _NUM_WORKERS == 0
  per_worker = batch // _NUM_WORKERS
  chunks = per_worker
  assert chunks % _NBUF == 0 and chunks >= _NBUF
  groups = chunks // _NBUF

  mesh = plsc.VectorSubcoreMesh(
      core_axis_name="c", subcore_axis_name="s",
      num_cores=_NUM_CORES, num_subcores=_NUM_SUBCORES)

  @functools.partial(
      pl.kernel,
      out_type=jax.ShapeDtypeStruct((batch, seq, hidden), jnp.float32),
      mesh=mesh,
      scratch_types=[
          pltpu.VMEM((per_worker, seq), jnp.int32),
          pltpu.VMEM((_NBUF, seq, hidden), jnp.float32),
          [pltpu.SemaphoreType.DMA] * _NBUF,
          [pltpu.SemaphoreType.DMA] * _NBUF,
      ],
      compiler_params=pltpu.CompilerParams(use_tc_tiling_on_sc=False),
  )
  def gather_kernel(table_hbm, idx_hbm, out_hbm, idx_v, rows_v, gsems, osems):
    wid = lax.axis_index("s") * _NUM_CORES + lax.axis_index("c")
    base = wid * per_worker

    pltpu.sync_copy(idx_hbm.at[pl.ds(base, per_worker)], idx_v)

    def gather_start(j, buf):
      pltpu.async_copy(table_hbm.at[idx_v.at[j]], rows_v.at[buf], gsems[buf])

    def gather_wait(j, buf):
      pltpu.make_async_copy(table_hbm.at[idx_v.at[j]], rows_v.at[buf],
                            gsems[buf]).wait()

    def out_start(j, buf):
      pltpu.async_copy(rows_v.at[buf], out_hbm.at[base + j], osems[buf])

    def out_wait(buf):
      pltpu.make_async_copy(rows_v.at[buf], out_hbm.at[base], osems[buf]).wait()

    for k in range(_AHEAD):
      gather_start(k, k)

    def body(g, _):
      for b in range(_NBUF):
        j = g * _NBUF + b
        nb = (b + _AHEAD) % _NBUF

        @pl.when(j + _AHEAD < chunks)
        def _():
          @pl.when(j + _AHEAD >= _NBUF)
          def _():
            out_wait(nb)
          gather_start(j + _AHEAD, nb)

        gather_wait(j, b)
        out_start(j, b)
      return ()

    lax.fori_loop(0, groups, body, ())

    for c in range(chunks - _NBUF, chunks):
      out_wait(c % _NBUF)

  return gather_kernel


def kernel(inputs, embeddings):
  batch, seq = inputs.shape
  hidden = embeddings.shape[1]
  return _make_gather(batch, seq, hidden)(embeddings, inputs.astype(jnp.int32))


# speedup vs baseline: 1.7085x; 1.0078x over previous
import functools

import jax
import jax.numpy as jnp
from jax import lax
from jax.experimental import pallas as pl
from jax.experimental.pallas import tpu as pltpu
from jax.experimental.pallas import tpu_sc as plsc

_NUM_CORES = 2
_NUM_SUBCORES = 16
_NUM_WORKERS = _NUM_CORES * _NUM_SUBCORES

_CHUNK = 128

_NBUF = 5
_AHEAD = 3


def _make_gather(total, hidden):
  assert total % (_NUM_WORKERS * _CHUNK) == 0
  per_worker = total // _NUM_WORKERS
  chunks = per_worker // _CHUNK
  assert chunks % _NBUF == 0 and chunks >= _NBUF
  groups = chunks // _NBUF

  mesh = plsc.VectorSubcoreMesh(
      core_axis_name="c", subcore_axis_name="s",
      num_cores=_NUM_CORES, num_subcores=_NUM_SUBCORES)

  @functools.partial(
      pl.kernel,
      out_type=jax.ShapeDtypeStruct((total, hidden), jnp.float32),
      mesh=mesh,
      scratch_types=[
          pltpu.VMEM((per_worker,), jnp.int32),
          pltpu.VMEM((_NBUF, _CHUNK, hidden), jnp.float32),
          [pltpu.SemaphoreType.DMA] * _NBUF,
          [pltpu.SemaphoreType.DMA] * _NBUF,
      ],
      compiler_params=pltpu.CompilerParams(use_tc_tiling_on_sc=False),
  )
  def gather_kernel(table_hbm, idx_hbm, out_hbm, idx_v, rows_v, gsems, osems):
    wid = lax.axis_index("s") * _NUM_CORES + lax.axis_index("c")
    base = wid * per_worker

    pltpu.sync_copy(idx_hbm.at[pl.ds(base, per_worker)], idx_v)

    def gather_start(j, buf):
      pltpu.async_copy(table_hbm.at[idx_v.at[pl.ds(j * _CHUNK, _CHUNK)]],
                       rows_v.at[buf], gsems[buf])

    def gather_wait(j, buf):
      pltpu.make_async_copy(table_hbm.at[idx_v.at[pl.ds(j * _CHUNK, _CHUNK)]],
                            rows_v.at[buf], gsems[buf]).wait()

    def out_start(j, buf):
      pltpu.async_copy(rows_v.at[buf],
                       out_hbm.at[pl.ds(base + j * _CHUNK, _CHUNK)],
                       osems[buf])

    def out_wait(buf):
      pltpu.make_async_copy(rows_v.at[buf],
                            out_hbm.at[pl.ds(base, _CHUNK)],
                            osems[buf]).wait()

    for k in range(_AHEAD):
      gather_start(k, k)

    def body(g, _):
      for b in range(_NBUF):
        j = g * _NBUF + b
        nb = (b + _AHEAD) % _NBUF

        @pl.when(j + _AHEAD < chunks)
        def _():
          @pl.when(j + _AHEAD >= _NBUF)
          def _():
            out_wait(nb)
          gather_start(j + _AHEAD, nb)

        gather_wait(j, b)
        out_start(j, b)
      return ()

    lax.fori_loop(0, groups, body, ())

    for c in range(chunks - _NBUF, chunks):
      out_wait(c % _NBUF)

  return gather_kernel


def kernel(inputs, embeddings):
  batch, seq = inputs.shape
  hidden = embeddings.shape[1]
  total = batch * seq
  flat_idx = jnp.reshape(inputs.astype(jnp.int32), (total,))
  out = _make_gather(total, hidden)(embeddings, flat_idx)
  return jnp.reshape(out, (batch, seq, hidden))
